# baseline (speedup 1.0000x reference)
import jax
import jax.numpy as jnp
from jax.experimental import pallas as pl
from jax.experimental.pallas import tpu as pltpu

N_TOK = 16384
HID = 1024
IBM = 1024


def _outer(x_hbm, w_ref, b_ref, o_hbm):
    w = w_ref[...]
    bias = b_ref[...]

    def body(x_blk, o_blk):
        acc = jax.lax.dot_general(
            x_blk[...].astype(jnp.bfloat16), w,
            dimension_numbers=(((1,), (1,)), ((), ())),
            preferred_element_type=jnp.float32,
        )
        o_blk[...] = acc + bias

    pltpu.emit_pipeline(
        body,
        grid=(N_TOK // IBM,),
        in_specs=[pl.BlockSpec((IBM, HID), lambda i: (i, 0))],
        out_specs=[pl.BlockSpec((IBM, HID), lambda i: (i, 0))],
    )(x_hbm, o_hbm)


def kernel(inputs, routing_vectors, W, b):
    orig_shape = inputs.shape
    x = inputs.reshape(-1, orig_shape[-1])
    w0 = W[0].astype(jnp.bfloat16)
    b0 = b[0].reshape(1, HID)

    out = pl.pallas_call(
        _outer,
        in_specs=[
            pl.BlockSpec(memory_space=pltpu.MemorySpace.HBM),
            pl.BlockSpec(memory_space=pltpu.MemorySpace.VMEM),
            pl.BlockSpec(memory_space=pltpu.MemorySpace.VMEM),
        ],
        out_specs=pl.BlockSpec(memory_space=pltpu.MemorySpace.HBM),
        out_shape=jax.ShapeDtypeStruct((N_TOK, HID), jnp.float32),
    )(x, w0, b0)
    return out.reshape(orig_shape)


# speedup vs baseline: 1.0575x; 1.0575x over previous
import jax
import jax.numpy as jnp
from jax.experimental import pallas as pl
from jax.experimental.pallas import tpu as pltpu

N_TOK = 16384
HID = 1024
IBM = 2048


def _outer(x_hbm, w_ref, b_ref, o_hbm):
    w = w_ref[...]
    bias = b_ref[...]

    def body(x_blk, o_blk):
        acc = jax.lax.dot_general(
            x_blk[...].astype(jnp.bfloat16), w,
            dimension_numbers=(((1,), (1,)), ((), ())),
            preferred_element_type=jnp.float32,
        )
        o_blk[...] = acc + bias

    pltpu.emit_pipeline(
        body,
        grid=(N_TOK // IBM,),
        in_specs=[pl.BlockSpec((IBM, HID), lambda i: (i, 0))],
        out_specs=[pl.BlockSpec((IBM, HID), lambda i: (i, 0))],
    )(x_hbm, o_hbm)


def kernel(inputs, routing_vectors, W, b):
    orig_shape = inputs.shape
    x = inputs.reshape(-1, orig_shape[-1])
    w0 = W[0].astype(jnp.bfloat16)
    b0 = b[0].reshape(1, HID)

    out = pl.pallas_call(
        _outer,
        in_specs=[
            pl.BlockSpec(memory_space=pltpu.MemorySpace.HBM),
            pl.BlockSpec(memory_space=pltpu.MemorySpace.VMEM),
            pl.BlockSpec(memory_space=pltpu.MemorySpace.VMEM),
        ],
        out_specs=pl.BlockSpec(memory_space=pltpu.MemorySpace.HBM),
        out_shape=jax.ShapeDtypeStruct((N_TOK, HID), jnp.float32),
    )(x, w0, b0)
    return out.reshape(orig_shape)


# speedup vs baseline: 1.1236x; 1.0625x over previous
import jax
import jax.numpy as jnp
from jax.experimental import pallas as pl
from jax.experimental.pallas import tpu as pltpu

N_TOK = 16384
HID = 1024
IBM = 2048


def _outer(x_hbm, w_ref, b_ref, o_hbm):
    w = w_ref[...]
    bias = b_ref[...]

    def body(x_blk, o_blk):
        acc = jax.lax.dot_general(
            x_blk[...].astype(jnp.bfloat16), w,
            dimension_numbers=(((1,), (1,)), ((), ())),
            preferred_element_type=jnp.float32,
        )
        o_blk[...] = acc + bias

    pltpu.emit_pipeline(
        body,
        grid=(N_TOK // IBM,),
        in_specs=[pl.BlockSpec((IBM, HID), lambda i: (i, 0),
                               pipeline_mode=pl.Buffered(buffer_count=3))],
        out_specs=[pl.BlockSpec((IBM, HID), lambda i: (i, 0))],
    )(x_hbm, o_hbm)


def kernel(inputs, routing_vectors, W, b):
    orig_shape = inputs.shape
    x = inputs.reshape(-1, orig_shape[-1])
    w0 = W[0].astype(jnp.bfloat16)
    b0 = b[0].reshape(1, HID)

    out = pl.pallas_call(
        _outer,
        in_specs=[
            pl.BlockSpec(memory_space=pltpu.MemorySpace.HBM),
            pl.BlockSpec(memory_space=pltpu.MemorySpace.VMEM),
            pl.BlockSpec(memory_space=pltpu.MemorySpace.VMEM),
        ],
        out_specs=pl.BlockSpec(memory_space=pltpu.MemorySpace.HBM),
        out_shape=jax.ShapeDtypeStruct((N_TOK, HID), jnp.float32),
    )(x, w0, b0)
    return out.reshape(orig_shape)


# speedup vs baseline: 1.1249x; 1.0012x over previous
import jax
import jax.numpy as jnp
from jax.experimental import pallas as pl
from jax.experimental.pallas import tpu as pltpu

N_TOK = 16384
HID = 1024
IBM = 1024


def _outer(x_hbm, w_ref, b_ref, o_hbm):
    w = w_ref[...]
    bias = b_ref[...]

    def body(x_blk, o_blk):
        acc = jax.lax.dot_general(
            x_blk[...].astype(jnp.bfloat16), w,
            dimension_numbers=(((1,), (1,)), ((), ())),
            preferred_element_type=jnp.float32,
        )
        o_blk[...] = acc + bias

    pltpu.emit_pipeline(
        body,
        grid=(N_TOK // IBM,),
        in_specs=[pl.BlockSpec((IBM, HID), lambda i: (i, 0),
                               pipeline_mode=pl.Buffered(buffer_count=6, use_lookahead=True))],
        out_specs=[pl.BlockSpec((IBM, HID), lambda i: (i, 0))],
    )(x_hbm, o_hbm)


def kernel(inputs, routing_vectors, W, b):
    orig_shape = inputs.shape
    x = inputs.reshape(-1, orig_shape[-1])
    w0 = W[0].astype(jnp.bfloat16)
    b0 = b[0].reshape(1, HID)

    out = pl.pallas_call(
        _outer,
        in_specs=[
            pl.BlockSpec(memory_space=pltpu.MemorySpace.HBM),
            pl.BlockSpec(memory_space=pltpu.MemorySpace.VMEM),
            pl.BlockSpec(memory_space=pltpu.MemorySpace.VMEM),
        ],
        out_specs=pl.BlockSpec(memory_space=pltpu.MemorySpace.HBM),
        out_shape=jax.ShapeDtypeStruct((N_TOK, HID), jnp.float32),
    )(x, w0, b0)
    return out.reshape(orig_shape)
